# baseline (speedup 1.0000x reference)
import functools

import jax
import jax.numpy as jnp
from jax import lax
from jax.experimental import pallas as pl
from jax.experimental.pallas import tpu as pltpu
from jax.experimental.pallas import tpu_sc as plsc

BATCH = 16384
DIM = 64
NW = 32
BPW = BATCH // NW
NU = 1000001
NBLK = (NU + 127) // 128
BLK_PW = (NBLK + NW - 1) // NW
NRING = 8


def _extract_phase(tab_t, sv_v, sp_v, st_v, out_hbm, bbuf, orow,
                   sems, bsem, w, slot_base):
  lane = lax.iota(jnp.int32, 16)
  j_lo = w * BLK_PW
  j_hi = jnp.minimum(j_lo + BLK_PW, NBLK)
  n_ent = st_v[pl.ds(j_hi, 16)][0] - st_v[pl.ds(j_lo, 16)][0]

  def block_body(jj, done):
    j = j_lo + jj

    def do_block():
      buf = bbuf.at[jj % 2]
      col0 = pl.multiple_of(j * 128, 128)
      pltpu.async_copy(tab_t.at[:, pl.ds(col0, 128)], buf, bsem).wait()
      lo = st_v[pl.ds(j, 16)][0]
      hi = st_v[pl.ds(j + 1, 16)][0]

      def ent_body(i, c2):
        e = lo + i
        val = sv_v[pl.ds(e, 16)][0]
        pos = sp_v[pl.ds(e, 16)][0]
        col = jnp.broadcast_to(val & 127, (16,))
        slot = (done + i) % NRING
        row = orow.at[slot]

        @pl.when(done + i >= NRING)
        def _():
          pltpu.make_async_copy(
              row, out_hbm.at[pl.ds(0, 1)], sems.at[slot]).wait()

        for c in range(4):
          vals = plsc.load_gather(buf, [lane + 16 * c, col])
          row[0, pl.ds(16 * c, 16)] = vals
        pltpu.async_copy(
            row, out_hbm.at[pl.ds(slot_base + pos, 1)], sems.at[slot])
        return c2

      lax.fori_loop(0, hi - lo, ent_body, 0)
      return hi - lo

    n = lax.cond(j < NBLK, do_block, lambda: jnp.int32(0))
    return done + n

  lax.fori_loop(0, BLK_PW, block_body, jnp.int32(0))

  def drain(s, c):
    @pl.when(s < jnp.minimum(n_ent, NRING))
    def _():
      pltpu.make_async_copy(
          orow.at[s], out_hbm.at[pl.ds(0, 1)], sems.at[s]).wait()
    return c

  lax.fori_loop(0, NRING, drain, 0)


def _sc_extract(u_sv, u_sp, u_st, i_sv, i_sp, i_st, ut_hbm, it_hbm, out_hbm,
                sv_v, sp_v, ust_v, ist_v, bbuf, orow, sems, bsem):
  w = lax.axis_index("s") * 2 + lax.axis_index("c")
  pltpu.sync_copy(u_st, ust_v)
  pltpu.sync_copy(i_st, ist_v)
  pltpu.sync_copy(u_sv, sv_v.at[pl.ds(0, BATCH + 16)])
  pltpu.sync_copy(u_sp, sp_v.at[pl.ds(0, BATCH + 16)])
  _extract_phase(ut_hbm, sv_v, sp_v, ust_v, out_hbm, bbuf, orow,
                 sems, bsem, w, 0)
  pltpu.sync_copy(i_sv, sv_v)
  pltpu.sync_copy(i_sp, sp_v)
  _extract_phase(it_hbm, sv_v, sp_v, ist_v, out_hbm, bbuf, orow,
                 sems, bsem, w, BATCH)


@jax.jit
def _sc_stage1(u_sv, u_sp, u_st, i_sv, i_sp, i_st, ut_t, it_t):
  mesh = plsc.VectorSubcoreMesh(core_axis_name="c", subcore_axis_name="s")
  f = pl.kernel(
      _sc_extract,
      out_type=jax.ShapeDtypeStruct((3 * BATCH, DIM), jnp.float32),
      mesh=mesh,
      compiler_params=pltpu.CompilerParams(
          needs_layout_passes=False, use_tc_tiling_on_sc=True,
          disable_bounds_checks=True),
      scratch_types=[
          pltpu.VMEM((2 * BATCH + 16,), jnp.int32),
          pltpu.VMEM((2 * BATCH + 16,), jnp.int32),
          pltpu.VMEM((NBLK + 17,), jnp.int32),
          pltpu.VMEM((NBLK + 17,), jnp.int32),
          pltpu.VMEM((2, DIM, 128), jnp.float32),
          pltpu.VMEM((NRING, 1, DIM), jnp.float32),
          pltpu.SemaphoreType.DMA((NRING,)),
          pltpu.SemaphoreType.DMA,
      ],
  )
  return f(u_sv, u_sp, u_st, i_sv, i_sp, i_st, ut_t, it_t)


def _sc_dots(rows_hbm, x_hbm, pu_v, qi_v, qj_v, x_v, sem):
  wid = lax.axis_index("s") * 2 + lax.axis_index("c")
  base = wid * BPW
  CH = 256

  def chunk_body(k, carry):
    ro = k * CH
    cp = [
        pltpu.async_copy(rows_hbm.at[pl.ds(base + ro, CH)], pu_v, sem),
        pltpu.async_copy(
            rows_hbm.at[pl.ds(BATCH + base + ro, CH)], qi_v, sem),
        pltpu.async_copy(
            rows_hbm.at[pl.ds(2 * BATCH + base + ro, CH)], qj_v, sem),
    ]
    for c in cp:
      c.wait()

    lane = lax.iota(jnp.int32, 16)

    def body(g, c2):
      acc16 = jnp.zeros((16,), jnp.float32)
      for r in range(16):
        row = g * 16 + r
        s = jnp.zeros((16,), jnp.float32)
        for c in range(4):
          cs = pl.ds(c * 16, 16)
          s = s + pu_v[row, cs] * (qi_v[row, cs] - qj_v[row, cs])
        acc16 = jnp.where(lane == r, jnp.sum(s), acc16)
      x_v[pl.ds(ro + g * 16, 16)] = acc16
      return c2

    lax.fori_loop(0, CH // 16, body, 0)
    return carry

  lax.fori_loop(0, BPW // CH, chunk_body, 0)
  pltpu.sync_copy(x_v, x_hbm.at[pl.ds(base, BPW)])


@jax.jit
def _sc_stage2(rows):
  mesh = plsc.VectorSubcoreMesh(core_axis_name="c", subcore_axis_name="s")
  f = pl.kernel(
      _sc_dots,
      out_type=jax.ShapeDtypeStruct((BATCH,), jnp.float32),
      mesh=mesh,
      compiler_params=pltpu.CompilerParams(
          needs_layout_passes=False, use_tc_tiling_on_sc=True),
      scratch_types=[
          pltpu.VMEM((256, DIM), jnp.float32),
          pltpu.VMEM((256, DIM), jnp.float32),
          pltpu.VMEM((256, DIM), jnp.float32),
          pltpu.VMEM((BPW,), jnp.float32),
          pltpu.SemaphoreType.DMA,
      ],
  )
  return f(rows)


def _loss_body(x_ref, out_ref):
  x = x_ref[...]
  t = jnp.log(jax.nn.sigmoid(x) + 1e-08)
  out_ref[0, 0] = -jnp.sum(t) * (1.0 / BATCH)


@jax.jit
def _tc_loss(x):
  res = pl.pallas_call(
      _loss_body,
      out_shape=jax.ShapeDtypeStruct((1, 1), jnp.float32),
      out_specs=pl.BlockSpec(memory_space=pltpu.SMEM),
  )(x.reshape(128, 128))
  return res[0, 0]


def _prep(idx, n):
  idx = idx.astype(jnp.int32)
  order = jnp.argsort(idx).astype(jnp.int32)
  sv = idx[order]
  edges = (jnp.arange(NBLK + 1, dtype=jnp.int32) * 128).astype(jnp.int32)
  starts = jnp.searchsorted(sv, edges, side="left").astype(jnp.int32)
  sv_p = jnp.pad(sv, (0, 16))
  sp_p = jnp.pad(order, (0, 16))
  st_p = jnp.pad(starts, (0, 16))
  return sv_p, sp_p, st_p


def kernel(u, i_pos, i_neg, user_table, item_table):
  u_sv, u_sp, u_st = _prep(u, BATCH)
  iv = jnp.concatenate([i_pos.astype(jnp.int32), i_neg.astype(jnp.int32)])
  i_sv, i_sp, i_st = _prep(iv, 2 * BATCH)
  rows = _sc_stage1(u_sv, u_sp, u_st, i_sv, i_sp, i_st,
                    user_table.T, item_table.T)
  x = _sc_stage2(rows)
  return _tc_loss(x)


# speedup vs baseline: 5.4743x; 5.4743x over previous
import functools

import jax
import jax.numpy as jnp
from jax import lax
from jax.experimental import pallas as pl
from jax.experimental.pallas import tpu as pltpu
from jax.experimental.pallas import tpu_sc as plsc

BATCH = 16384
DIM = 64
NW = 32
BPW = BATCH // NW
NU = 1000001
NBLK = (NU + 127) // 128
BLK_PW = (NBLK + NW - 1) // NW
NRING = 8
NBUF = 4


def _extract_phase(tab_t, sv_v, sp_v, st_v, out_hbm, bbuf, orow,
                   sems, bsem, w, slot_base):
  lane = lax.iota(jnp.int32, 16)
  j_lo = w * BLK_PW
  j_hi = jnp.minimum(j_lo + BLK_PW, NBLK)
  nblk_w = j_hi - j_lo
  n_ent = st_v[pl.ds(j_hi, 16)][0] - st_v[pl.ds(j_lo, 16)][0]

  def issue(jj):
    @pl.when(jj < nblk_w)
    def _():
      col0 = pl.multiple_of((j_lo + jj) * 128, 128)
      pltpu.async_copy(tab_t.at[:, pl.ds(col0, 128)],
                       bbuf.at[jj % NBUF], bsem.at[jj % NBUF])

  for p in range(NBUF - 1):
    issue(jnp.int32(p))

  def block_body(jj, done):
    j = j_lo + jj

    def do_block():
      buf = bbuf.at[jj % NBUF]
      pltpu.make_async_copy(tab_t.at[:, pl.ds(0, 128)], buf,
                            bsem.at[jj % NBUF]).wait()
      issue(jj + NBUF - 1)
      lo = st_v[pl.ds(j, 16)][0]
      hi = st_v[pl.ds(j + 1, 16)][0]

      def ent_body(i, c2):
        e = lo + i
        val = sv_v[pl.ds(e, 16)][0]
        pos = sp_v[pl.ds(e, 16)][0]
        col = jnp.broadcast_to(val & 127, (16,))
        slot = (done + i) % NRING
        row = orow.at[slot]

        @pl.when(done + i >= NRING)
        def _():
          pltpu.make_async_copy(
              row, out_hbm.at[pl.ds(0, 1)], sems.at[slot]).wait()

        for c in range(4):
          vals = plsc.load_gather(buf, [lane + 16 * c, col])
          row[0, pl.ds(16 * c, 16)] = vals
        pltpu.async_copy(
            row, out_hbm.at[pl.ds(slot_base + pos, 1)], sems.at[slot])
        return c2

      lax.fori_loop(0, hi - lo, ent_body, 0)
      return hi - lo

    n = lax.cond(j < NBLK, do_block, lambda: jnp.int32(0))
    return done + n

  lax.fori_loop(0, BLK_PW, block_body, jnp.int32(0))

  def drain(s, c):
    @pl.when(s < jnp.minimum(n_ent, NRING))
    def _():
      pltpu.make_async_copy(
          orow.at[s], out_hbm.at[pl.ds(0, 1)], sems.at[s]).wait()
    return c

  lax.fori_loop(0, NRING, drain, 0)


def _sc_extract(u_sv, u_sp, u_st, i_sv, i_sp, i_st, ut_hbm, it_hbm, out_hbm,
                sv_v, sp_v, ust_v, ist_v, bbuf, orow, sems, bsem):
  w = lax.axis_index("s") * 2 + lax.axis_index("c")
  pltpu.sync_copy(u_st, ust_v)
  pltpu.sync_copy(i_st, ist_v)
  pltpu.sync_copy(u_sv, sv_v.at[pl.ds(0, BATCH + 16)])
  pltpu.sync_copy(u_sp, sp_v.at[pl.ds(0, BATCH + 16)])
  _extract_phase(ut_hbm, sv_v, sp_v, ust_v, out_hbm, bbuf, orow,
                 sems, bsem, w, 0)
  pltpu.sync_copy(i_sv, sv_v)
  pltpu.sync_copy(i_sp, sp_v)
  _extract_phase(it_hbm, sv_v, sp_v, ist_v, out_hbm, bbuf, orow,
                 sems, bsem, w, BATCH)


@jax.jit
def _sc_stage1(u_sv, u_sp, u_st, i_sv, i_sp, i_st, ut_t, it_t):
  mesh = plsc.VectorSubcoreMesh(core_axis_name="c", subcore_axis_name="s")
  f = pl.kernel(
      _sc_extract,
      out_type=jax.ShapeDtypeStruct((3 * BATCH, DIM), jnp.float32),
      mesh=mesh,
      compiler_params=pltpu.CompilerParams(
          needs_layout_passes=False, use_tc_tiling_on_sc=True,
          disable_bounds_checks=True),
      scratch_types=[
          pltpu.VMEM((2 * BATCH + 16,), jnp.int32),
          pltpu.VMEM((2 * BATCH + 16,), jnp.int32),
          pltpu.VMEM((NBLK + 17,), jnp.int32),
          pltpu.VMEM((NBLK + 17,), jnp.int32),
          pltpu.VMEM((NBUF, DIM, 128), jnp.float32),
          pltpu.VMEM((NRING, 1, DIM), jnp.float32),
          pltpu.SemaphoreType.DMA((NRING,)),
          pltpu.SemaphoreType.DMA((NBUF,)),
      ],
  )
  return f(u_sv, u_sp, u_st, i_sv, i_sp, i_st, ut_t, it_t)


def _sc_dots(rows_hbm, x_hbm, pu_v, qi_v, qj_v, x_v, sem):
  wid = lax.axis_index("s") * 2 + lax.axis_index("c")
  base = wid * BPW
  CH = 256

  def chunk_body(k, carry):
    ro = k * CH
    cp = [
        pltpu.async_copy(rows_hbm.at[pl.ds(base + ro, CH)], pu_v, sem),
        pltpu.async_copy(
            rows_hbm.at[pl.ds(BATCH + base + ro, CH)], qi_v, sem),
        pltpu.async_copy(
            rows_hbm.at[pl.ds(2 * BATCH + base + ro, CH)], qj_v, sem),
    ]
    for c in cp:
      c.wait()

    lane = lax.iota(jnp.int32, 16)

    def body(g, c2):
      acc16 = jnp.zeros((16,), jnp.float32)
      for r in range(16):
        row = g * 16 + r
        s = jnp.zeros((16,), jnp.float32)
        for c in range(4):
          cs = pl.ds(c * 16, 16)
          s = s + pu_v[row, cs] * (qi_v[row, cs] - qj_v[row, cs])
        acc16 = jnp.where(lane == r, jnp.sum(s), acc16)
      x_v[pl.ds(ro + g * 16, 16)] = acc16
      return c2

    lax.fori_loop(0, CH // 16, body, 0)
    return carry

  lax.fori_loop(0, BPW // CH, chunk_body, 0)
  pltpu.sync_copy(x_v, x_hbm.at[pl.ds(base, BPW)])


@jax.jit
def _sc_stage2(rows):
  mesh = plsc.VectorSubcoreMesh(core_axis_name="c", subcore_axis_name="s")
  f = pl.kernel(
      _sc_dots,
      out_type=jax.ShapeDtypeStruct((BATCH,), jnp.float32),
      mesh=mesh,
      compiler_params=pltpu.CompilerParams(
          needs_layout_passes=False, use_tc_tiling_on_sc=True),
      scratch_types=[
          pltpu.VMEM((256, DIM), jnp.float32),
          pltpu.VMEM((256, DIM), jnp.float32),
          pltpu.VMEM((256, DIM), jnp.float32),
          pltpu.VMEM((BPW,), jnp.float32),
          pltpu.SemaphoreType.DMA,
      ],
  )
  return f(rows)


def _loss_body(x_ref, out_ref):
  x = x_ref[...]
  t = jnp.log(jax.nn.sigmoid(x) + 1e-08)
  out_ref[0, 0] = -jnp.sum(t) * (1.0 / BATCH)


@jax.jit
def _tc_loss(x):
  res = pl.pallas_call(
      _loss_body,
      out_shape=jax.ShapeDtypeStruct((1, 1), jnp.float32),
      out_specs=pl.BlockSpec(memory_space=pltpu.SMEM),
  )(x.reshape(128, 128))
  return res[0, 0]


def _prep(idx, n):
  idx = idx.astype(jnp.int32)
  order = jnp.argsort(idx).astype(jnp.int32)
  sv = idx[order]
  counts = jnp.zeros((NBLK,), jnp.int32).at[sv >> 7].add(1)
  starts = jnp.concatenate(
      [jnp.zeros((1,), jnp.int32), jnp.cumsum(counts).astype(jnp.int32)])
  sv_p = jnp.pad(sv, (0, 16))
  sp_p = jnp.pad(order, (0, 16))
  st_p = jnp.pad(starts, (0, 16))
  return sv_p, sp_p, st_p


def kernel(u, i_pos, i_neg, user_table, item_table):
  u_sv, u_sp, u_st = _prep(u, BATCH)
  iv = jnp.concatenate([i_pos.astype(jnp.int32), i_neg.astype(jnp.int32)])
  i_sv, i_sp, i_st = _prep(iv, 2 * BATCH)
  rows = _sc_stage1(u_sv, u_sp, u_st, i_sv, i_sp, i_st,
                    user_table.T, item_table.T)
  x = _sc_stage2(rows)
  return _tc_loss(x)


# speedup vs baseline: 5.9781x; 1.0920x over previous
import functools

import jax
import jax.numpy as jnp
from jax import lax
from jax.experimental import pallas as pl
from jax.experimental.pallas import tpu as pltpu
from jax.experimental.pallas import tpu_sc as plsc

BATCH = 16384
DIM = 64
NW = 32
BPW = BATCH // NW
NU = 1000001
NBLK = (NU + 127) // 128
BLK_PW = (NBLK + NW - 1) // NW
NRING = 8
NBUF = 4


def _extract_phase(tab_t, sv_v, sp_v, st_v, out_hbm, bbuf, orow,
                   sems, bsem, w, slot_base):
  lane = lax.iota(jnp.int32, 16)
  j_lo = w * BLK_PW
  j_hi = jnp.minimum(j_lo + BLK_PW, NBLK)
  nblk_w = j_hi - j_lo
  n_ent = st_v[pl.ds(j_hi, 16)][0] - st_v[pl.ds(j_lo, 16)][0]

  def issue(jj):
    @pl.when(jj < nblk_w)
    def _():
      col0 = pl.multiple_of((j_lo + jj) * 128, 128)
      pltpu.async_copy(tab_t.at[:, pl.ds(col0, 128)],
                       bbuf.at[jj % NBUF], bsem.at[jj % NBUF])

  for p in range(NBUF - 1):
    issue(jnp.int32(p))

  def block_body(jj, done):
    j = j_lo + jj

    def do_block():
      buf = bbuf.at[jj % NBUF]
      pltpu.make_async_copy(tab_t.at[:, pl.ds(0, 128)], buf,
                            bsem.at[jj % NBUF]).wait()
      issue(jj + NBUF - 1)
      lo = st_v[pl.ds(j, 16)][0]
      hi = st_v[pl.ds(j + 1, 16)][0]

      def ent_body(i, c2):
        e = lo + i
        val = sv_v[pl.ds(e, 16)][0]
        pos = sp_v[pl.ds(e, 16)][0]
        col = jnp.broadcast_to(val & 127, (16,))
        slot = (done + i) % NRING
        row = orow.at[slot]

        @pl.when(done + i >= NRING)
        def _():
          pltpu.make_async_copy(
              row, out_hbm.at[pl.ds(0, 1)], sems.at[slot]).wait()

        for c in range(4):
          vals = plsc.load_gather(buf, [lane + 16 * c, col])
          row[0, pl.ds(16 * c, 16)] = vals
        pltpu.async_copy(
            row, out_hbm.at[pl.ds(slot_base + pos, 1)], sems.at[slot])
        return c2

      lax.fori_loop(0, hi - lo, ent_body, 0)
      return hi - lo

    n = lax.cond(j < NBLK, do_block, lambda: jnp.int32(0))
    return done + n

  lax.fori_loop(0, BLK_PW, block_body, jnp.int32(0))

  def drain(s, c):
    @pl.when(s < jnp.minimum(n_ent, NRING))
    def _():
      pltpu.make_async_copy(
          orow.at[s], out_hbm.at[pl.ds(0, 1)], sems.at[s]).wait()
    return c

  lax.fori_loop(0, NRING, drain, 0)


def _sc_extract(u_sv, u_sp, u_st, i_sv, i_sp, i_st, ut_hbm, it_hbm, out_hbm,
                sv_v, sp_v, ust_v, ist_v, bbuf, orow, sems, bsem):
  w = lax.axis_index("s") * 2 + lax.axis_index("c")
  pltpu.sync_copy(u_st, ust_v)
  pltpu.sync_copy(i_st, ist_v)
  pltpu.sync_copy(u_sv, sv_v.at[pl.ds(0, BATCH + 16)])
  pltpu.sync_copy(u_sp, sp_v.at[pl.ds(0, BATCH + 16)])
  _extract_phase(ut_hbm, sv_v, sp_v, ust_v, out_hbm, bbuf, orow,
                 sems, bsem, w, 0)
  pltpu.sync_copy(i_sv, sv_v)
  pltpu.sync_copy(i_sp, sp_v)
  _extract_phase(it_hbm, sv_v, sp_v, ist_v, out_hbm, bbuf, orow,
                 sems, bsem, w, BATCH)


@jax.jit
def _sc_stage1(u_sv, u_sp, u_st, i_sv, i_sp, i_st, ut_t, it_t):
  mesh = plsc.VectorSubcoreMesh(core_axis_name="c", subcore_axis_name="s")
  f = pl.kernel(
      _sc_extract,
      out_type=jax.ShapeDtypeStruct((3 * BATCH, DIM), jnp.float32),
      mesh=mesh,
      compiler_params=pltpu.CompilerParams(
          needs_layout_passes=False, use_tc_tiling_on_sc=True,
          disable_bounds_checks=True),
      scratch_types=[
          pltpu.VMEM((2 * BATCH + 16,), jnp.int32),
          pltpu.VMEM((2 * BATCH + 16,), jnp.int32),
          pltpu.VMEM((NBLK + 17,), jnp.int32),
          pltpu.VMEM((NBLK + 17,), jnp.int32),
          pltpu.VMEM((NBUF, DIM, 128), jnp.float32),
          pltpu.VMEM((NRING, 1, DIM), jnp.float32),
          pltpu.SemaphoreType.DMA((NRING,)),
          pltpu.SemaphoreType.DMA((NBUF,)),
      ],
  )
  return f(u_sv, u_sp, u_st, i_sv, i_sp, i_st, ut_t, it_t)


def _loss_body(rows_ref, out_ref):
  pu = rows_ref[0:BATCH, :]
  qi = rows_ref[BATCH:2 * BATCH, :]
  qj = rows_ref[2 * BATCH:3 * BATCH, :]
  x = jnp.sum(pu * (qi - qj), axis=1)
  t = jnp.log(jax.nn.sigmoid(x) + 1e-08)
  out_ref[0, 0] = -jnp.sum(t) * (1.0 / BATCH)


@jax.jit
def _tc_loss(rows):
  res = pl.pallas_call(
      _loss_body,
      out_shape=jax.ShapeDtypeStruct((1, 1), jnp.float32),
      out_specs=pl.BlockSpec(memory_space=pltpu.SMEM),
  )(rows)
  return res[0, 0]


def _prep(idx, n):
  idx = idx.astype(jnp.int32)
  iota = jnp.arange(idx.shape[0], dtype=jnp.int32)
  sv, order = lax.sort((idx, iota), num_keys=1)
  counts = jnp.zeros((NBLK,), jnp.int32).at[sv >> 7].add(1)
  starts = jnp.concatenate(
      [jnp.zeros((1,), jnp.int32), jnp.cumsum(counts).astype(jnp.int32)])
  sv_p = jnp.pad(sv, (0, 16))
  sp_p = jnp.pad(order, (0, 16))
  st_p = jnp.pad(starts, (0, 16))
  return sv_p, sp_p, st_p


def kernel(u, i_pos, i_neg, user_table, item_table):
  u_sv, u_sp, u_st = _prep(u, BATCH)
  iv = jnp.concatenate([i_pos.astype(jnp.int32), i_neg.astype(jnp.int32)])
  i_sv, i_sp, i_st = _prep(iv, 2 * BATCH)
  rows = _sc_stage1(u_sv, u_sp, u_st, i_sv, i_sp, i_st,
                    user_table.T, item_table.T)
  return _tc_loss(rows)


# speedup vs baseline: 6.1937x; 1.0361x over previous
import functools

import jax
import jax.numpy as jnp
from jax import lax
from jax.experimental import pallas as pl
from jax.experimental.pallas import tpu as pltpu
from jax.experimental.pallas import tpu_sc as plsc

BATCH = 16384
DIM = 64
NW = 32
BPW = BATCH // NW
NU = 1000001
NBLK = (NU + 127) // 128
BLK_PW = (NBLK + NW - 1) // NW
NRING = 8
NBUF = 4


def _extract_phase(tab_t, sv_v, sp_v, st_v, out_hbm, bbuf, orow,
                   sems, bsem, w, slot_base):
  lane = lax.iota(jnp.int32, 16)
  j_lo = w * BLK_PW
  j_hi = jnp.minimum(j_lo + BLK_PW, NBLK)
  nblk_w = j_hi - j_lo
  n_ent = st_v[pl.ds(j_hi, 16)][0] - st_v[pl.ds(j_lo, 16)][0]

  def nonempty(jj):
    j = j_lo + jj
    return st_v[pl.ds(j + 1, 16)][0] > st_v[pl.ds(j, 16)][0]

  def issue(jj):
    @pl.when(jnp.logical_and(jj < nblk_w, nonempty(jj)))
    def _():
      col0 = pl.multiple_of((j_lo + jj) * 128, 128)
      pltpu.async_copy(tab_t.at[:, pl.ds(col0, 128)],
                       bbuf.at[jj % NBUF], bsem.at[jj % NBUF])

  for p in range(NBUF - 1):
    issue(jnp.int32(p))

  def block_body(jj, done):
    j = j_lo + jj

    def do_block():
      buf = bbuf.at[jj % NBUF]
      issue(jj + NBUF - 1)
      lo = st_v[pl.ds(j, 16)][0]
      hi = st_v[pl.ds(j + 1, 16)][0]

      @pl.when(hi > lo)
      def _():
        pltpu.make_async_copy(tab_t.at[:, pl.ds(0, 128)], buf,
                              bsem.at[jj % NBUF]).wait()

      def ent_body(i, c2):
        e = lo + i
        val = sv_v[pl.ds(e, 16)][0]
        pos = sp_v[pl.ds(e, 16)][0]
        col = jnp.broadcast_to(val & 127, (16,))
        slot = (done + i) % NRING
        row = orow.at[slot]

        @pl.when(done + i >= NRING)
        def _():
          pltpu.make_async_copy(
              row, out_hbm.at[pl.ds(0, 1)], sems.at[slot]).wait()

        for c in range(4):
          vals = plsc.load_gather(buf, [lane + 16 * c, col])
          row[0, pl.ds(16 * c, 16)] = vals
        pltpu.async_copy(
            row, out_hbm.at[pl.ds(slot_base + pos, 1)], sems.at[slot])
        return c2

      lax.fori_loop(0, hi - lo, ent_body, 0)
      return hi - lo

    n = lax.cond(j < NBLK, do_block, lambda: jnp.int32(0))
    return done + n

  lax.fori_loop(0, BLK_PW, block_body, jnp.int32(0))

  def drain(s, c):
    @pl.when(s < jnp.minimum(n_ent, NRING))
    def _():
      pltpu.make_async_copy(
          orow.at[s], out_hbm.at[pl.ds(0, 1)], sems.at[s]).wait()
    return c

  lax.fori_loop(0, NRING, drain, 0)


def _sc_extract(u_sv, u_sp, u_st, i_sv, i_sp, i_st, ut_hbm, it_hbm, out_hbm,
                sv_v, sp_v, ust_v, ist_v, bbuf, orow, sems, bsem):
  w = lax.axis_index("s") * 2 + lax.axis_index("c")
  pltpu.sync_copy(u_st, ust_v)
  pltpu.sync_copy(i_st, ist_v)
  pltpu.sync_copy(u_sv, sv_v.at[pl.ds(0, BATCH + 16)])
  pltpu.sync_copy(u_sp, sp_v.at[pl.ds(0, BATCH + 16)])
  _extract_phase(ut_hbm, sv_v, sp_v, ust_v, out_hbm, bbuf, orow,
                 sems, bsem, w, 0)
  pltpu.sync_copy(i_sv, sv_v)
  pltpu.sync_copy(i_sp, sp_v)
  _extract_phase(it_hbm, sv_v, sp_v, ist_v, out_hbm, bbuf, orow,
                 sems, bsem, w, BATCH)


@jax.jit
def _sc_stage1(u_sv, u_sp, u_st, i_sv, i_sp, i_st, ut_t, it_t):
  mesh = plsc.VectorSubcoreMesh(core_axis_name="c", subcore_axis_name="s")
  f = pl.kernel(
      _sc_extract,
      out_type=jax.ShapeDtypeStruct((3 * BATCH, DIM), jnp.float32),
      mesh=mesh,
      compiler_params=pltpu.CompilerParams(
          needs_layout_passes=False, use_tc_tiling_on_sc=True,
          disable_bounds_checks=True),
      scratch_types=[
          pltpu.VMEM((2 * BATCH + 16,), jnp.int32),
          pltpu.VMEM((2 * BATCH + 16,), jnp.int32),
          pltpu.VMEM((NBLK + 17,), jnp.int32),
          pltpu.VMEM((NBLK + 17,), jnp.int32),
          pltpu.VMEM((NBUF, DIM, 128), jnp.float32),
          pltpu.VMEM((NRING, 1, DIM), jnp.float32),
          pltpu.SemaphoreType.DMA((NRING,)),
          pltpu.SemaphoreType.DMA((NBUF,)),
      ],
  )
  return f(u_sv, u_sp, u_st, i_sv, i_sp, i_st, ut_t, it_t)


def _loss_body(rows_ref, out_ref):
  pu = rows_ref[0:BATCH, :]
  qi = rows_ref[BATCH:2 * BATCH, :]
  qj = rows_ref[2 * BATCH:3 * BATCH, :]
  x = jnp.sum(pu * (qi - qj), axis=1)
  t = jnp.log(jax.nn.sigmoid(x) + 1e-08)
  out_ref[0, 0] = -jnp.sum(t) * (1.0 / BATCH)


@jax.jit
def _tc_loss(rows):
  res = pl.pallas_call(
      _loss_body,
      out_shape=jax.ShapeDtypeStruct((1, 1), jnp.float32),
      out_specs=pl.BlockSpec(memory_space=pltpu.SMEM),
  )(rows)
  return res[0, 0]


def _prep(idx, n):
  idx = idx.astype(jnp.int32)
  iota = jnp.arange(idx.shape[0], dtype=jnp.int32)
  sv, order = lax.sort((idx, iota), num_keys=1)
  counts = jnp.zeros((NBLK,), jnp.int32).at[sv >> 7].add(1)
  starts = jnp.concatenate(
      [jnp.zeros((1,), jnp.int32), jnp.cumsum(counts).astype(jnp.int32)])
  sv_p = jnp.pad(sv, (0, 16))
  sp_p = jnp.pad(order, (0, 16))
  st_p = jnp.pad(starts, (0, 16))
  return sv_p, sp_p, st_p


def kernel(u, i_pos, i_neg, user_table, item_table):
  u_sv, u_sp, u_st = _prep(u, BATCH)
  iv = jnp.concatenate([i_pos.astype(jnp.int32), i_neg.astype(jnp.int32)])
  i_sv, i_sp, i_st = _prep(iv, 2 * BATCH)
  rows = _sc_stage1(u_sv, u_sp, u_st, i_sv, i_sp, i_st,
                    user_table.T, item_table.T)
  return _tc_loss(rows)
